# baseline (speedup 1.0000x reference)
import functools

import jax
import jax.numpy as jnp
from jax import lax
from jax.experimental import pallas as pl
from jax.experimental.pallas import tpu as pltpu
from jax.experimental.pallas import tpu_sc as plsc

NUM_EXPERTS = 8
TOP_K = 2
D_MODEL = 1024
D_FF = 512
T = 2048

BROW = 256
NBLK = (T * TOP_K) // BROW + NUM_EXPERTS
PAD = NBLK * BROW

NW = 32


def _router_body(x_ref, gate_ref, pos1_ref, pos2_ref, w1_ref, w2_ref, ebl_ref):
    x = x_ref[...]
    logits = jnp.dot(x, gate_ref[...], preferred_element_type=jnp.float32)
    m = jnp.max(logits, axis=-1, keepdims=True)
    ex = jnp.exp(logits - m)
    probs = ex / jnp.sum(ex, axis=-1, keepdims=True)

    idx = lax.broadcasted_iota(jnp.int32, probs.shape, 1)
    m1 = jnp.max(probs, axis=-1, keepdims=True)
    i1 = jnp.min(jnp.where(probs == m1, idx, NUM_EXPERTS), axis=-1, keepdims=True)
    mask1 = idx == i1
    probs2 = jnp.where(mask1, -jnp.inf, probs)
    m2 = jnp.max(probs2, axis=-1, keepdims=True)
    i2 = jnp.min(jnp.where(probs2 == m2, idx, NUM_EXPERTS), axis=-1, keepdims=True)
    mask2 = idx == i2

    w1_ref[...] = jnp.sum(jnp.where(mask1, probs, 0.0), axis=-1, keepdims=True)
    w2_ref[...] = jnp.sum(jnp.where(mask2, probs, 0.0), axis=-1, keepdims=True)

    one1 = jnp.where(mask1, 1.0, 0.0)
    one2 = jnp.where(mask2, 1.0, 0.0)

    r_io = lax.broadcasted_iota(jnp.int32, (T, T), 0)
    c_io = lax.broadcasted_iota(jnp.int32, (T, T), 1)
    tril = jnp.where(c_io < r_io, 1.0, 0.0).astype(jnp.bfloat16)
    cum1 = jnp.dot(tril, one1.astype(jnp.bfloat16), preferred_element_type=jnp.float32)
    cum2 = jnp.dot(tril, one2.astype(jnp.bfloat16), preferred_element_type=jnp.float32)

    cnt1 = jnp.sum(one1, axis=0, keepdims=True)
    cnt2 = jnp.sum(one2, axis=0, keepdims=True)
    cnt = cnt1 + cnt2
    nb = jnp.floor((cnt + (BROW - 1)) * (1.0 / BROW))

    e_r = lax.broadcasted_iota(jnp.int32, (NUM_EXPERTS, NUM_EXPERTS), 0)
    e_c = lax.broadcasted_iota(jnp.int32, (NUM_EXPERTS, NUM_EXPERTS), 1)
    emat = jnp.where(e_r < e_c, 1.0, 0.0)
    off = jnp.dot(nb, emat, preferred_element_type=jnp.float32) * float(BROW)

    pos1 = jnp.sum(jnp.where(mask1, off + cum1, 0.0), axis=-1, keepdims=True)
    pos2 = jnp.sum(jnp.where(mask2, off + cnt1 + cum2, 0.0), axis=-1, keepdims=True)
    pos1_ref[...] = pos1.astype(jnp.int32)
    pos2_ref[...] = pos2.astype(jnp.int32)

    b_io = lax.broadcasted_iota(jnp.int32, (NBLK, NUM_EXPERTS), 0).astype(jnp.float32)
    started = jnp.where(off <= b_io * float(BROW), 1.0, 0.0)
    ebl_ref[...] = (jnp.sum(started, axis=-1, keepdims=True) - 1.0).astype(jnp.int32)


def _run_router(x, gate_w):
    return pl.pallas_call(
        _router_body,
        grid=(1,),
        in_specs=[
            pl.BlockSpec((T, D_MODEL), lambda i: (0, 0)),
            pl.BlockSpec((D_MODEL, NUM_EXPERTS), lambda i: (0, 0)),
        ],
        out_specs=[
            pl.BlockSpec((T, 1), lambda i: (0, 0)),
            pl.BlockSpec((T, 1), lambda i: (0, 0)),
            pl.BlockSpec((T, 1), lambda i: (0, 0)),
            pl.BlockSpec((T, 1), lambda i: (0, 0)),
            pl.BlockSpec((NBLK, 1), lambda i: (0, 0)),
        ],
        out_shape=[
            jax.ShapeDtypeStruct((T, 1), jnp.int32),
            jax.ShapeDtypeStruct((T, 1), jnp.int32),
            jax.ShapeDtypeStruct((T, 1), jnp.float32),
            jax.ShapeDtypeStruct((T, 1), jnp.float32),
            jax.ShapeDtypeStruct((NBLK, 1), jnp.int32),
        ],
    )(x, gate_w)


def _make_scatter():
    mesh = plsc.VectorSubcoreMesh(core_axis_name="c", subcore_axis_name="s")

    @functools.partial(
        pl.kernel,
        mesh=mesh,
        compiler_params=pltpu.CompilerParams(needs_layout_passes=False),
        out_type=[
            jax.ShapeDtypeStruct((PAD,), jnp.int32),
            jax.ShapeDtypeStruct((PAD,), jnp.float32),
        ],
        scratch_types=[
            pltpu.VMEM((T,), jnp.int32),
            pltpu.VMEM((T,), jnp.int32),
            pltpu.VMEM((T,), jnp.float32),
            pltpu.VMEM((T,), jnp.float32),
            pltpu.VMEM((PAD,), jnp.int32),
            pltpu.VMEM((PAD,), jnp.float32),
        ],
    )
    def scatter_k(pos1_h, pos2_h, w1_h, w2_h, tok_h, ws_h,
                  pos1_v, pos2_v, w1_v, w2_v, tok_v, ws_v):
        wid = lax.axis_index("s") * 2 + lax.axis_index("c")

        @pl.when(wid == 0)
        def _():
            pltpu.sync_copy(pos1_h, pos1_v)
            pltpu.sync_copy(pos2_h, pos2_v)
            pltpu.sync_copy(w1_h, w1_v)
            pltpu.sync_copy(w2_h, w2_v)

            zi = jnp.zeros((16,), jnp.int32)
            zf = jnp.zeros((16,), jnp.float32)

            def zero_body(i, _):
                tok_v[pl.ds(i * 16, 16)] = zi
                ws_v[pl.ds(i * 16, 16)] = zf
                return 0
            lax.fori_loop(0, PAD // 16, zero_body, 0)

            lane = lax.iota(jnp.int32, 16)

            def scat_body(c, _):
                tvals = lane + c * 16
                p1 = pos1_v[pl.ds(c * 16, 16)]
                plsc.store_scatter(tok_v, [p1], tvals)
                plsc.store_scatter(ws_v, [p1], w1_v[pl.ds(c * 16, 16)])
                p2 = pos2_v[pl.ds(c * 16, 16)]
                plsc.store_scatter(tok_v, [p2], tvals)
                plsc.store_scatter(ws_v, [p2], w2_v[pl.ds(c * 16, 16)])
                return 0
            lax.fori_loop(0, T // 16, scat_body, 0)

            pltpu.sync_copy(tok_v, tok_h)
            pltpu.sync_copy(ws_v, ws_h)

    return scatter_k


def _make_gather():
    mesh = plsc.VectorSubcoreMesh(core_axis_name="c", subcore_axis_name="s")
    rows_w = PAD // NW
    chunk = rows_w // 2
    wvec = D_MODEL // 2

    @functools.partial(
        pl.kernel,
        mesh=mesh,
        compiler_params=pltpu.CompilerParams(needs_layout_passes=False),
        out_type=jax.ShapeDtypeStruct((PAD, wvec), jnp.int32),
        scratch_types=[
            pltpu.VMEM((chunk,), jnp.int32),
            pltpu.VMEM((chunk, wvec), jnp.int32),
            pltpu.SemaphoreType.DMA,
        ],
    )
    def gather_k(xbits_h, tok_h, xs_h, idx_v, rows_v, sem):
        wid = lax.axis_index("s") * 2 + lax.axis_index("c")
        base = wid * rows_w

        def do(c, _):
            pltpu.sync_copy(tok_h.at[pl.ds(base + c * chunk, chunk)], idx_v)
            pltpu.async_copy(xbits_h.at[idx_v], rows_v, sem).wait()
            pltpu.sync_copy(rows_v, xs_h.at[pl.ds(base + c * chunk, chunk)])
            return 0
        lax.fori_loop(0, 2, do, 0)

    return gather_k


def _ffn_body(ebl_ref, xs_ref, ws_ref, wgu_ref, wdn_ref, ys_ref):
    xb = xs_ref[...]
    gu = jnp.dot(xb, wgu_ref[0], preferred_element_type=jnp.float32)
    g = gu[:, :D_FF]
    u = gu[:, D_FF:]
    act = (g * lax.logistic(g)) * u * ws_ref[...]
    ys_ref[...] = jnp.dot(act.astype(jnp.bfloat16), wdn_ref[0],
                          preferred_element_type=jnp.float32)


def _run_ffn(ebl, xs_bf16, ws2d, wgu, wdn):
    grid_spec = pltpu.PrefetchScalarGridSpec(
        num_scalar_prefetch=1,
        grid=(NBLK,),
        in_specs=[
            pl.BlockSpec((BROW, D_MODEL), lambda i, ebl: (i, 0)),
            pl.BlockSpec((BROW, 1), lambda i, ebl: (i, 0)),
            pl.BlockSpec((1, D_MODEL, 2 * D_FF), lambda i, ebl: (ebl[i], 0, 0)),
            pl.BlockSpec((1, D_FF, D_MODEL), lambda i, ebl: (ebl[i], 0, 0)),
        ],
        out_specs=pl.BlockSpec((BROW, D_MODEL), lambda i, ebl: (i, 0)),
    )
    return pl.pallas_call(
        _ffn_body,
        grid_spec=grid_spec,
        out_shape=jax.ShapeDtypeStruct((PAD, D_MODEL), jnp.float32),
        compiler_params=pltpu.CompilerParams(
            dimension_semantics=("arbitrary",),
        ),
    )(ebl, xs_bf16, ws2d, wgu, wdn)


def _make_combine():
    mesh = plsc.VectorSubcoreMesh(core_axis_name="c", subcore_axis_name="s")
    tok_w = T // NW
    chunk = tok_w // 2
    nvec = D_MODEL // 16

    @functools.partial(
        pl.kernel,
        mesh=mesh,
        compiler_params=pltpu.CompilerParams(needs_layout_passes=False),
        out_type=jax.ShapeDtypeStruct((T, D_MODEL), jnp.float32),
        scratch_types=[
            pltpu.VMEM((chunk,), jnp.int32),
            pltpu.VMEM((chunk,), jnp.int32),
            pltpu.VMEM((chunk, D_MODEL), jnp.float32),
            pltpu.VMEM((chunk, D_MODEL), jnp.float32),
            pltpu.SemaphoreType.DMA,
            pltpu.SemaphoreType.DMA,
        ],
    )
    def combine_k(ys_h, pos1_h, pos2_h, out_h,
                  idx1_v, idx2_v, buf1, buf2, sem1, sem2):
        wid = lax.axis_index("s") * 2 + lax.axis_index("c")
        base = wid * tok_w

        def do(c, _):
            tb = base + c * chunk
            pltpu.sync_copy(pos1_h.at[pl.ds(tb, chunk)], idx1_v)
            pltpu.sync_copy(pos2_h.at[pl.ds(tb, chunk)], idx2_v)
            cp1 = pltpu.async_copy(ys_h.at[idx1_v], buf1, sem1)
            cp2 = pltpu.async_copy(ys_h.at[idx2_v], buf2, sem2)
            cp1.wait()
            cp2.wait()

            def add_row(r, _):
                def add_vec(v, _):
                    for q in range(8):
                        sl = pl.ds((v * 8 + q) * 16, 16)
                        buf1[r, sl] = buf1[r, sl] + buf2[r, sl]
                    return 0
                lax.fori_loop(0, nvec // 8, add_vec, 0)
                return 0
            lax.fori_loop(0, chunk, add_row, 0)

            pltpu.sync_copy(buf1, out_h.at[pl.ds(tb, chunk)])
            return 0
        lax.fori_loop(0, 2, do, 0)

    return combine_k


def kernel(hidden_states, num_global_tokens, max_num_tokens_per_gpu, gate_w, w_gate_up, w_down):
    x = hidden_states
    wgu = w_gate_up.astype(jnp.bfloat16)
    wdn = w_down.astype(jnp.bfloat16)

    pos1, pos2, w1, w2, ebl = _run_router(x, gate_w)
    pos1f = pos1.reshape(T)
    pos2f = pos2.reshape(T)

    tok, ws = _make_scatter()(pos1f, pos2f, w1.reshape(T), w2.reshape(T))

    xbits = lax.bitcast_convert_type(
        x.astype(jnp.bfloat16).reshape(T, D_MODEL // 2, 2), jnp.int32)
    xs_bits = _make_gather()(xbits, tok)
    xs_bf16 = lax.bitcast_convert_type(xs_bits, jnp.bfloat16).reshape(PAD, D_MODEL)

    ys = _run_ffn(ebl.reshape(NBLK), xs_bf16, ws.reshape(PAD, 1), wgu, wdn)

    out = _make_combine()(ys, pos1f, pos2f)
    return out


# speedup vs baseline: 3.5066x; 3.5066x over previous
import jax
import jax.numpy as jnp
from jax import lax
from jax.experimental import pallas as pl
from jax.experimental.pallas import tpu as pltpu

NUM_EXPERTS = 8
TOP_K = 2
D_MODEL = 1024
D_FF = 512
T = 2048

BT = 256
EF = NUM_EXPERTS * D_FF


def _body(x_ref, gate_ref, wgu_ref, wdn_ref, out_ref):
    x = x_ref[...]

    logits = jnp.dot(x, gate_ref[...], preferred_element_type=jnp.float32)
    m = jnp.max(logits, axis=-1, keepdims=True)
    ex = jnp.exp(logits - m)
    probs = ex / jnp.sum(ex, axis=-1, keepdims=True)

    idx = lax.broadcasted_iota(jnp.int32, probs.shape, 1)
    m1 = jnp.max(probs, axis=-1, keepdims=True)
    i1 = jnp.min(jnp.where(probs == m1, idx, NUM_EXPERTS), axis=-1, keepdims=True)
    mask1 = idx == i1
    probs2 = jnp.where(mask1, -jnp.inf, probs)
    m2 = jnp.max(probs2, axis=-1, keepdims=True)
    i2 = jnp.min(jnp.where(probs2 == m2, idx, NUM_EXPERTS), axis=-1, keepdims=True)
    mask2 = idx == i2
    combine = jnp.where(mask1 | mask2, probs, 0.0)

    w_exp = jnp.broadcast_to(combine[:, :, None], (BT, NUM_EXPERTS, D_FF))
    w_exp = w_exp.reshape(BT, EF)

    xb = x.astype(jnp.bfloat16)
    gu = jnp.dot(xb, wgu_ref[...], preferred_element_type=jnp.float32)
    g = gu[:, :EF]
    u = gu[:, EF:]
    act = (g * lax.logistic(g)) * u * w_exp
    out_ref[...] = jnp.dot(act.astype(jnp.bfloat16), wdn_ref[...],
                           preferred_element_type=jnp.float32)


def kernel(hidden_states, num_global_tokens, max_num_tokens_per_gpu, gate_w, w_gate_up, w_down):
    wg = w_gate_up[:, :, :D_FF].transpose(1, 0, 2).reshape(D_MODEL, EF)
    wu = w_gate_up[:, :, D_FF:].transpose(1, 0, 2).reshape(D_MODEL, EF)
    wgu_cat = jnp.concatenate([wg, wu], axis=1).astype(jnp.bfloat16)
    wdn_cat = w_down.reshape(EF, D_MODEL).astype(jnp.bfloat16)

    nT = T // BT
    return pl.pallas_call(
        _body,
        grid=(nT,),
        in_specs=[
            pl.BlockSpec((BT, D_MODEL), lambda i: (i, 0)),
            pl.BlockSpec((D_MODEL, NUM_EXPERTS), lambda i: (0, 0)),
            pl.BlockSpec((D_MODEL, 2 * EF), lambda i: (0, 0)),
            pl.BlockSpec((EF, D_MODEL), lambda i: (0, 0)),
        ],
        out_specs=pl.BlockSpec((BT, D_MODEL), lambda i: (i, 0)),
        out_shape=jax.ShapeDtypeStruct((T, D_MODEL), jnp.float32),
        compiler_params=pltpu.CompilerParams(
            dimension_semantics=("arbitrary",),
        ),
    )(hidden_states, gate_w, wgu_cat, wdn_cat)
